# Initial kernel scaffold; baseline (speedup 1.0000x reference)
import jax
import jax.numpy as jnp
from jax.experimental import pallas as pl


def kernel(node_mars, element_mars, scratch, nids, cids):
    raise NotImplementedError("write your pallas kernel here")


# speedup vs baseline: 5.7491x; 5.7491x over previous
import jax
import jax.numpy as jnp
from jax import lax
from jax.experimental import pallas as pl
from jax.experimental.pallas import tpu as pltpu
from jax.experimental.pallas import tpu_sc as plsc

NUM_NODES = 200000
MAX_ELS = 200001
B = 128
NC = 2
NS = 16
NW = NC * NS
CHUNK_OUT = 64
CHUNK_IDX = 2 * CHUNK_OUT
NCHUNKS = NUM_NODES // CHUNK_OUT
CHUNKS_PER_W = -(-NCHUNKS // NW)
NFULL = NCHUNKS % NW
LANES = 16
DGROUPS = B // LANES


def _body(node_hbm, idx_hbm, em_hbm, out_hbm, idx_v, buf, out_v, row_v, gsem):
    c = lax.axis_index("c")
    s = lax.axis_index("s")
    wid = s * NC + c
    nchunks_w = jnp.where(wid < NFULL, CHUNKS_PER_W, CHUNKS_PER_W - 1)

    pltpu.sync_copy(idx_hbm.at[wid], idx_v)

    def chunk(t, carry):
        pltpu.async_copy(node_hbm.at[idx_v.at[t]], buf, gsem).wait()

        def row(j, carry2):
            for d in range(DGROUPS):
                sl = pl.ds(d * LANES, LANES)
                out_v[j, sl] = buf[2 * j, sl] + buf[2 * j + 1, sl]
            return carry2

        lax.fori_loop(0, CHUNK_OUT, row, 0, unroll=2)
        base = (wid + t * NW) * CHUNK_OUT
        pltpu.sync_copy(out_v, out_hbm.at[pl.ds(base, CHUNK_OUT)])
        return carry

    lax.fori_loop(0, nchunks_w, chunk, 0)

    @pl.when(wid == 0)
    def _():
        pltpu.sync_copy(em_hbm.at[pl.ds(NUM_NODES, 1)], row_v)
        pltpu.sync_copy(row_v, out_hbm.at[pl.ds(NUM_NODES, 1)])


@jax.jit
def _run(node_mars, element_mars, idx_all):
    mesh = plsc.VectorSubcoreMesh(
        core_axis_name="c", subcore_axis_name="s", num_cores=NC, num_subcores=NS
    )
    return pl.kernel(
        _body,
        out_type=jax.ShapeDtypeStruct((MAX_ELS, B), jnp.float32),
        mesh=mesh,
        scratch_types=[
            pltpu.VMEM((CHUNKS_PER_W, CHUNK_IDX), jnp.int32),
            pltpu.VMEM((CHUNK_IDX, B), jnp.float32),
            pltpu.VMEM((CHUNK_OUT, B), jnp.float32),
            pltpu.VMEM((1, B), jnp.float32),
            pltpu.SemaphoreType.DMA,
        ],
    )(node_mars, idx_all, element_mars)


def kernel(node_mars, element_mars, scratch, nids, cids):
    idx = cids.reshape(NCHUNKS, CHUNK_IDX)
    pad = NW * CHUNKS_PER_W - NCHUNKS
    idx = jnp.concatenate([idx, jnp.zeros((pad, CHUNK_IDX), jnp.int32)], axis=0)
    idx_all = idx.reshape(CHUNKS_PER_W, NW, CHUNK_IDX).transpose(1, 0, 2)
    return _run(node_mars, element_mars, idx_all)
